# baseline (speedup 1.0000x reference)
import jax
import jax.numpy as jnp
from jax.experimental import pallas as pl
from jax.experimental.pallas import tpu as pltpu

N_REVIEW = 100000
D_REVIEW = 799
H = 128
BN = 8192


def _fused_body(x_ref, w1_ref, b1_ref,
                w2st_ref, b2st_ref, w2sf_ref, b2sf_ref, w2sb_ref, b2sb_ref,
                wr_ref, br_ref, wi_ref, bi_ref, ws_ref, bs_ref,
                out_r_ref, out_i_ref, out_s_ref):
    bf = jnp.bfloat16
    xT = x_ref[...].astype(bf)
    a = jnp.dot(w1_ref[...].astype(bf), xT,
                preferred_element_type=jnp.float32)
    a = jnp.maximum(a + b1_ref[...].T, 0.0).astype(bf)
    x1_review = a[:H, :]
    x1_product = a[H:, :]

    x2r = jnp.maximum(
        jnp.dot(w2st_ref[...].T.astype(bf), x1_review,
                preferred_element_type=jnp.float32)
        + b2st_ref[...].T, 0.0).astype(bf)
    x2i = jnp.maximum(
        jnp.dot(w2sf_ref[...].T.astype(bf), x1_review,
                preferred_element_type=jnp.float32)
        + b2sf_ref[...].T, 0.0).astype(bf)
    x2s = jnp.maximum(
        jnp.dot(w2sb_ref[...].T.astype(bf), x1_product,
                preferred_element_type=jnp.float32)
        + b2sb_ref[...].T, 0.0).astype(bf)

    out_r_ref[...] = jax.nn.sigmoid(
        jnp.dot(wr_ref[...].T.astype(bf), x2r,
                preferred_element_type=jnp.float32) + br_ref[...])[0]
    out_i_ref[...] = jax.nn.sigmoid(
        jnp.dot(wi_ref[...].T.astype(bf), x2i,
                preferred_element_type=jnp.float32) + bi_ref[...])[0]
    out_s_ref[...] = jax.nn.sigmoid(
        jnp.dot(ws_ref[...].T.astype(bf), x2s,
                preferred_element_type=jnp.float32) + bs_ref[...])[0]


def kernel(x_review, x_product, edge_written_for, edge_sold_by, edge_sent_from,
           edge_similar_to,
           W1_wf, b1_wf, W1_sb, b1_sb, W1_sf, b1_sf, W1_st, b1_st,
           W2_wf, b2_wf, W2_sb, b2_sb, W2_sf, b2_sf, W2_st, b2_st,
           Wr, br, Wi, bi, Ws, bs):
    w1T = jnp.concatenate([W1_st.T, W1_wf.T], axis=0)
    b1 = jnp.concatenate([b1_st, b1_wf])[None, :]

    full = lambda shape: pl.BlockSpec(shape, lambda i: tuple(0 for _ in shape))
    grid = (N_REVIEW + BN - 1) // BN

    out_r, out_i, out_s = pl.pallas_call(
        _fused_body,
        grid=(grid,),
        in_specs=[
            pl.BlockSpec((D_REVIEW, BN), lambda i: (0, i)),
            full((2 * H, D_REVIEW)), full((1, 2 * H)),
            full((H, H)), full((1, H)),
            full((H, H)), full((1, H)),
            full((H, H)), full((1, H)),
            full((H, 1)), full((1, 1)),
            full((H, 1)), full((1, 1)),
            full((H, 1)), full((1, 1)),
        ],
        out_specs=[
            pl.BlockSpec((BN,), lambda i: (i,)),
            pl.BlockSpec((BN,), lambda i: (i,)),
            pl.BlockSpec((BN,), lambda i: (i,)),
        ],
        out_shape=[
            jax.ShapeDtypeStruct((N_REVIEW,), jnp.float32),
            jax.ShapeDtypeStruct((N_REVIEW,), jnp.float32),
            jax.ShapeDtypeStruct((N_REVIEW,), jnp.float32),
        ],
        compiler_params=pltpu.CompilerParams(
            dimension_semantics=("arbitrary",),
        ),
    )(x_review.T, w1T, b1,
      W2_st, b2_st[None, :], W2_sf, b2_sf[None, :], W2_sb, b2_sb[None, :],
      Wr, br[None, :], Wi, bi[None, :], Ws, bs[None, :])

    return (out_r, out_i, out_s)


# speedup vs baseline: 1.0145x; 1.0145x over previous
import jax
import jax.numpy as jnp
from jax.experimental import pallas as pl
from jax.experimental.pallas import tpu as pltpu

N_REVIEW = 100000
D_REVIEW = 799
H = 128
BN = 6144


def _fused_body(x_ref, w1_ref, b1_ref,
                w2st_ref, b2st_ref, w2sf_ref, b2sf_ref, w2sb_ref, b2sb_ref,
                wr_ref, br_ref, wi_ref, bi_ref, ws_ref, bs_ref,
                out_r_ref, out_i_ref, out_s_ref):
    bf = jnp.bfloat16
    xT = x_ref[...].astype(bf)
    a = jnp.dot(w1_ref[...].astype(bf), xT,
                preferred_element_type=jnp.float32)
    a = jnp.maximum(a + b1_ref[...].T, 0.0).astype(bf)
    x1_review = a[:H, :]
    x1_product = a[H:, :]

    x2r = jnp.maximum(
        jnp.dot(w2st_ref[...].T.astype(bf), x1_review,
                preferred_element_type=jnp.float32)
        + b2st_ref[...].T, 0.0).astype(bf)
    x2i = jnp.maximum(
        jnp.dot(w2sf_ref[...].T.astype(bf), x1_review,
                preferred_element_type=jnp.float32)
        + b2sf_ref[...].T, 0.0).astype(bf)
    x2s = jnp.maximum(
        jnp.dot(w2sb_ref[...].T.astype(bf), x1_product,
                preferred_element_type=jnp.float32)
        + b2sb_ref[...].T, 0.0).astype(bf)

    out_r_ref[...] = jax.nn.sigmoid(
        jnp.dot(wr_ref[...].T.astype(bf), x2r,
                preferred_element_type=jnp.float32) + br_ref[...])[0]
    out_i_ref[...] = jax.nn.sigmoid(
        jnp.dot(wi_ref[...].T.astype(bf), x2i,
                preferred_element_type=jnp.float32) + bi_ref[...])[0]
    out_s_ref[...] = jax.nn.sigmoid(
        jnp.dot(ws_ref[...].T.astype(bf), x2s,
                preferred_element_type=jnp.float32) + bs_ref[...])[0]


def kernel(x_review, x_product, edge_written_for, edge_sold_by, edge_sent_from,
           edge_similar_to,
           W1_wf, b1_wf, W1_sb, b1_sb, W1_sf, b1_sf, W1_st, b1_st,
           W2_wf, b2_wf, W2_sb, b2_sb, W2_sf, b2_sf, W2_st, b2_st,
           Wr, br, Wi, bi, Ws, bs):
    w1T = jnp.concatenate([W1_st.T, W1_wf.T], axis=0)
    b1 = jnp.concatenate([b1_st, b1_wf])[None, :]

    full = lambda shape: pl.BlockSpec(shape, lambda i: tuple(0 for _ in shape))
    grid = (N_REVIEW + BN - 1) // BN

    out_r, out_i, out_s = pl.pallas_call(
        _fused_body,
        grid=(grid,),
        in_specs=[
            pl.BlockSpec((D_REVIEW, BN), lambda i: (0, i)),
            full((2 * H, D_REVIEW)), full((1, 2 * H)),
            full((H, H)), full((1, H)),
            full((H, H)), full((1, H)),
            full((H, H)), full((1, H)),
            full((H, 1)), full((1, 1)),
            full((H, 1)), full((1, 1)),
            full((H, 1)), full((1, 1)),
        ],
        out_specs=[
            pl.BlockSpec((BN,), lambda i: (i,)),
            pl.BlockSpec((BN,), lambda i: (i,)),
            pl.BlockSpec((BN,), lambda i: (i,)),
        ],
        out_shape=[
            jax.ShapeDtypeStruct((N_REVIEW,), jnp.float32),
            jax.ShapeDtypeStruct((N_REVIEW,), jnp.float32),
            jax.ShapeDtypeStruct((N_REVIEW,), jnp.float32),
        ],
        compiler_params=pltpu.CompilerParams(
            dimension_semantics=("parallel",),
        ),
    )(x_review.T, w1T, b1,
      W2_st, b2_st[None, :], W2_sf, b2_sf[None, :], W2_sb, b2_sb[None, :],
      Wr, br[None, :], Wi, bi[None, :], Ws, bs[None, :])

    return (out_r, out_i, out_s)


# speedup vs baseline: 1.0176x; 1.0031x over previous
import jax
import jax.numpy as jnp
from jax.experimental import pallas as pl
from jax.experimental.pallas import tpu as pltpu

N_REVIEW = 100000
D_REVIEW = 799
H = 128
BN = 5120


def _fused_body(x_ref, w1_ref, b1_ref,
                w2st_ref, b2st_ref, w2sf_ref, b2sf_ref, w2sb_ref, b2sb_ref,
                wr_ref, br_ref, wi_ref, bi_ref, ws_ref, bs_ref,
                out_r_ref, out_i_ref, out_s_ref):
    bf = jnp.bfloat16
    xT = x_ref[...].astype(bf)
    a = jnp.dot(w1_ref[...].astype(bf), xT,
                preferred_element_type=jnp.float32)
    a = jnp.maximum(a + b1_ref[...].T, 0.0).astype(bf)
    x1_review = a[:H, :]
    x1_product = a[H:, :]

    x2r = jnp.maximum(
        jnp.dot(w2st_ref[...].T.astype(bf), x1_review,
                preferred_element_type=jnp.float32)
        + b2st_ref[...].T, 0.0).astype(bf)
    x2i = jnp.maximum(
        jnp.dot(w2sf_ref[...].T.astype(bf), x1_review,
                preferred_element_type=jnp.float32)
        + b2sf_ref[...].T, 0.0).astype(bf)
    x2s = jnp.maximum(
        jnp.dot(w2sb_ref[...].T.astype(bf), x1_product,
                preferred_element_type=jnp.float32)
        + b2sb_ref[...].T, 0.0).astype(bf)

    out_r_ref[...] = jax.nn.sigmoid(
        jnp.dot(wr_ref[...].T.astype(bf), x2r,
                preferred_element_type=jnp.float32) + br_ref[...])[0]
    out_i_ref[...] = jax.nn.sigmoid(
        jnp.dot(wi_ref[...].T.astype(bf), x2i,
                preferred_element_type=jnp.float32) + bi_ref[...])[0]
    out_s_ref[...] = jax.nn.sigmoid(
        jnp.dot(ws_ref[...].T.astype(bf), x2s,
                preferred_element_type=jnp.float32) + bs_ref[...])[0]


def kernel(x_review, x_product, edge_written_for, edge_sold_by, edge_sent_from,
           edge_similar_to,
           W1_wf, b1_wf, W1_sb, b1_sb, W1_sf, b1_sf, W1_st, b1_st,
           W2_wf, b2_wf, W2_sb, b2_sb, W2_sf, b2_sf, W2_st, b2_st,
           Wr, br, Wi, bi, Ws, bs):
    w1T = jnp.concatenate([W1_st.T, W1_wf.T], axis=0)
    b1 = jnp.concatenate([b1_st, b1_wf])[None, :]

    full = lambda shape: pl.BlockSpec(shape, lambda i: tuple(0 for _ in shape))
    grid = (N_REVIEW + BN - 1) // BN

    out_r, out_i, out_s = pl.pallas_call(
        _fused_body,
        grid=(grid,),
        in_specs=[
            pl.BlockSpec((D_REVIEW, BN), lambda i: (0, i)),
            full((2 * H, D_REVIEW)), full((1, 2 * H)),
            full((H, H)), full((1, H)),
            full((H, H)), full((1, H)),
            full((H, H)), full((1, H)),
            full((H, 1)), full((1, 1)),
            full((H, 1)), full((1, 1)),
            full((H, 1)), full((1, 1)),
        ],
        out_specs=[
            pl.BlockSpec((BN,), lambda i: (i,)),
            pl.BlockSpec((BN,), lambda i: (i,)),
            pl.BlockSpec((BN,), lambda i: (i,)),
        ],
        out_shape=[
            jax.ShapeDtypeStruct((N_REVIEW,), jnp.float32),
            jax.ShapeDtypeStruct((N_REVIEW,), jnp.float32),
            jax.ShapeDtypeStruct((N_REVIEW,), jnp.float32),
        ],
        compiler_params=pltpu.CompilerParams(
            dimension_semantics=("parallel",),
        ),
    )(x_review.T, w1T, b1,
      W2_st, b2_st[None, :], W2_sf, b2_sf[None, :], W2_sb, b2_sb[None, :],
      Wr, br[None, :], Wi, bi[None, :], Ws, bs[None, :])

    return (out_r, out_i, out_s)


# speedup vs baseline: 1.0523x; 1.0341x over previous
import jax
import jax.numpy as jnp
from jax.experimental import pallas as pl
from jax.experimental.pallas import tpu as pltpu

N_REVIEW = 100000
D_REVIEW = 799
H = 128
BN = 7168


def _fused_body(x_ref, w1_ref, b1_ref,
                w2st_ref, b2st_ref, w2sf_ref, b2sf_ref, w2sb_ref, b2sb_ref,
                wr_ref, br_ref, wi_ref, bi_ref, ws_ref, bs_ref,
                out_r_ref, out_i_ref, out_s_ref):
    bf = jnp.bfloat16
    xT = x_ref[...].astype(bf)
    a = jnp.dot(w1_ref[...].astype(bf), xT,
                preferred_element_type=jnp.float32)
    a = jnp.maximum(a + b1_ref[...].T, 0.0).astype(bf)
    x1_review = a[:H, :]
    x1_product = a[H:, :]

    x2r = jnp.maximum(
        jnp.dot(w2st_ref[...].T.astype(bf), x1_review,
                preferred_element_type=jnp.float32)
        + b2st_ref[...].T, 0.0).astype(bf)
    x2i = jnp.maximum(
        jnp.dot(w2sf_ref[...].T.astype(bf), x1_review,
                preferred_element_type=jnp.float32)
        + b2sf_ref[...].T, 0.0).astype(bf)
    x2s = jnp.maximum(
        jnp.dot(w2sb_ref[...].T.astype(bf), x1_product,
                preferred_element_type=jnp.float32)
        + b2sb_ref[...].T, 0.0).astype(bf)

    out_r_ref[...] = jax.nn.sigmoid(
        jnp.dot(wr_ref[...].T.astype(bf), x2r,
                preferred_element_type=jnp.float32) + br_ref[...])[0]
    out_i_ref[...] = jax.nn.sigmoid(
        jnp.dot(wi_ref[...].T.astype(bf), x2i,
                preferred_element_type=jnp.float32) + bi_ref[...])[0]
    out_s_ref[...] = jax.nn.sigmoid(
        jnp.dot(ws_ref[...].T.astype(bf), x2s,
                preferred_element_type=jnp.float32) + bs_ref[...])[0]


def kernel(x_review, x_product, edge_written_for, edge_sold_by, edge_sent_from,
           edge_similar_to,
           W1_wf, b1_wf, W1_sb, b1_sb, W1_sf, b1_sf, W1_st, b1_st,
           W2_wf, b2_wf, W2_sb, b2_sb, W2_sf, b2_sf, W2_st, b2_st,
           Wr, br, Wi, bi, Ws, bs):
    w1T = jnp.concatenate([W1_st.T, W1_wf.T], axis=0)
    b1 = jnp.concatenate([b1_st, b1_wf])[None, :]

    full = lambda shape: pl.BlockSpec(shape, lambda i: tuple(0 for _ in shape))
    grid = (N_REVIEW + BN - 1) // BN

    out_r, out_i, out_s = pl.pallas_call(
        _fused_body,
        grid=(grid,),
        in_specs=[
            pl.BlockSpec((D_REVIEW, BN), lambda i: (0, i)),
            full((2 * H, D_REVIEW)), full((1, 2 * H)),
            full((H, H)), full((1, H)),
            full((H, H)), full((1, H)),
            full((H, H)), full((1, H)),
            full((H, 1)), full((1, 1)),
            full((H, 1)), full((1, 1)),
            full((H, 1)), full((1, 1)),
        ],
        out_specs=[
            pl.BlockSpec((BN,), lambda i: (i,)),
            pl.BlockSpec((BN,), lambda i: (i,)),
            pl.BlockSpec((BN,), lambda i: (i,)),
        ],
        out_shape=[
            jax.ShapeDtypeStruct((N_REVIEW,), jnp.float32),
            jax.ShapeDtypeStruct((N_REVIEW,), jnp.float32),
            jax.ShapeDtypeStruct((N_REVIEW,), jnp.float32),
        ],
        compiler_params=pltpu.CompilerParams(
            dimension_semantics=("parallel",),
        ),
    )(x_review.T, w1T, b1,
      W2_st, b2_st[None, :], W2_sf, b2_sf[None, :], W2_sb, b2_sb[None, :],
      Wr, br[None, :], Wi, bi[None, :], Ws, bs[None, :])

    return (out_r, out_i, out_s)
